# Initial kernel scaffold; baseline (speedup 1.0000x reference)
import jax
import jax.numpy as jnp
from jax.experimental import pallas as pl


def kernel(atomic_numbers, index_map):
    raise NotImplementedError("write your pallas kernel here")


# speedup vs baseline: 5.4391x; 5.4391x over previous
import functools

import jax
import jax.numpy as jnp
from jax import lax
from jax.experimental import pallas as pl
from jax.experimental.pallas import tpu as pltpu
from jax.experimental.pallas import tpu_sc as plsc

NUM_CLASSES = 119
N_ATOMS = 100000
L = 16
R = 400
G = N_ATOMS // R
CW = R * NUM_CLASSES
IMAP_PAD = 128


def kernel(atomic_numbers, index_map):
    info = plsc.get_sparse_core_info()
    nc, ns = info.num_cores, info.num_subcores
    nw = nc * ns

    imap_pad = jnp.pad(index_map, (0, IMAP_PAD - NUM_CLASSES))

    mesh = plsc.VectorSubcoreMesh(core_axis_name="c", subcore_axis_name="s")

    @functools.partial(
        pl.kernel,
        out_type=jax.ShapeDtypeStruct((N_ATOMS * NUM_CLASSES,), jnp.float32),
        mesh=mesh,
        scratch_types=[
            pltpu.VMEM((IMAP_PAD,), jnp.int32),
            pltpu.VMEM((R,), jnp.int32),
            pltpu.VMEM((R,), jnp.int32),
            pltpu.VMEM((CW,), jnp.float32),
        ],
        compiler_params=pltpu.CompilerParams(needs_layout_passes=False),
    )
    def sc_kernel(a_hbm, imap_hbm, out_hbm, imap_v, a_v, idx_save, buf):
        wid = lax.axis_index("s") * nc + lax.axis_index("c")
        pltpu.sync_copy(imap_hbm, imap_v)

        zeros16 = jnp.zeros((L,), jnp.float32)
        ones16 = jnp.full((L,), 1.0, jnp.float32)
        row16 = lax.iota(jnp.int32, L)

        def zero_body(i, carry):
            buf[pl.ds(i * L, L)] = zeros16
            return carry

        lax.fori_loop(0, CW // L, zero_body, 0)

        niter = (G - wid + nw - 1) // nw

        def chunk_body(t, carry):
            g = wid + t * nw
            pltpu.sync_copy(a_hbm.at[pl.ds(g * R, R)], a_v)
            for j in range(R // L):
                av = a_v[pl.ds(j * L, L)]
                mapped = plsc.load_gather(imap_v, [av])
                flat = (row16 + (j * L)) * NUM_CLASSES + mapped
                plsc.store_scatter(buf, [flat], ones16)
                idx_save[pl.ds(j * L, L)] = flat
            pltpu.sync_copy(buf, out_hbm.at[pl.ds(g * CW, CW)])
            for j in range(R // L):
                flat = idx_save[pl.ds(j * L, L)]
                plsc.store_scatter(buf, [flat], zeros16)
            return carry

        lax.fori_loop(0, niter, chunk_body, 0)

    out_flat = sc_kernel(atomic_numbers, imap_pad)
    return out_flat.reshape(N_ATOMS, NUM_CLASSES)


# speedup vs baseline: 11.9943x; 2.2052x over previous
import functools

import jax
import jax.numpy as jnp
from jax import lax
from jax.experimental import pallas as pl
from jax.experimental.pallas import tpu as pltpu
from jax.experimental.pallas import tpu_sc as plsc

NUM_CLASSES = 119
N_ATOMS = 100000
L = 16
R = 400
G = N_ATOMS // R
IMAP_PAD = 128


def kernel(atomic_numbers, index_map):
    info = plsc.get_sparse_core_info()
    nc, ns = info.num_cores, info.num_subcores
    nw = nc * ns

    imap_pad = jnp.pad(index_map, (0, IMAP_PAD - NUM_CLASSES))

    mesh = plsc.VectorSubcoreMesh(core_axis_name="c", subcore_axis_name="s")

    @functools.partial(
        pl.kernel,
        out_type=jax.ShapeDtypeStruct((N_ATOMS, NUM_CLASSES), jnp.float32),
        mesh=mesh,
        scratch_types=[
            pltpu.VMEM((IMAP_PAD,), jnp.int32),
            pltpu.VMEM((R,), jnp.int32),
            pltpu.VMEM((R,), jnp.int32),
            pltpu.VMEM((R, NUM_CLASSES), jnp.float32),
        ],
        compiler_params=pltpu.CompilerParams(needs_layout_passes=False),
    )
    def sc_kernel(a_hbm, imap_hbm, out_hbm, imap_v, a_v, col_save, buf):
        wid = lax.axis_index("s") * nc + lax.axis_index("c")
        pltpu.sync_copy(imap_hbm, imap_v)

        zeros16 = jnp.zeros((L,), jnp.float32)
        ones16 = jnp.full((L,), 1.0, jnp.float32)
        row16 = lax.iota(jnp.int32, L)

        tail_cols = row16 + (NUM_CLASSES // L) * L
        tail_mask = tail_cols < NUM_CLASSES

        def zero_body(r, carry):
            for j in range(NUM_CLASSES // L):
                buf[r, pl.ds(j * L, L)] = zeros16
            plsc.store_scatter(
                buf, [jnp.full((L,), r, jnp.int32), tail_cols], zeros16,
                mask=tail_mask)
            return carry

        lax.fori_loop(0, R, zero_body, 0)

        niter = (G - wid + nw - 1) // nw

        def chunk_body(t, carry):
            g = wid + t * nw
            pltpu.sync_copy(a_hbm.at[pl.ds(g * R, R)], a_v)
            for j in range(R // L):
                av = a_v[pl.ds(j * L, L)]
                mapped = plsc.load_gather(imap_v, [av])
                plsc.store_scatter(buf, [row16 + (j * L), mapped], ones16)
                col_save[pl.ds(j * L, L)] = mapped
            pltpu.sync_copy(buf, out_hbm.at[pl.ds(g * R, R)])
            for j in range(R // L):
                mapped = col_save[pl.ds(j * L, L)]
                plsc.store_scatter(buf, [row16 + (j * L), mapped], zeros16)
            return carry

        lax.fori_loop(0, niter, chunk_body, 0)

    return sc_kernel(atomic_numbers, imap_pad)


# speedup vs baseline: 12.0249x; 1.0026x over previous
import functools

import jax
import jax.numpy as jnp
from jax import lax
from jax.experimental import pallas as pl
from jax.experimental.pallas import tpu as pltpu
from jax.experimental.pallas import tpu_sc as plsc

NUM_CLASSES = 119
N_ATOMS = 100000
L = 16
R = 400
G = N_ATOMS // R
IMAP_PAD = 128


def kernel(atomic_numbers, index_map):
    info = plsc.get_sparse_core_info()
    nc, ns = info.num_cores, info.num_subcores
    nw = nc * ns

    imap_pad = jnp.pad(index_map, (0, IMAP_PAD - NUM_CLASSES))

    mesh = plsc.VectorSubcoreMesh(core_axis_name="c", subcore_axis_name="s")

    @functools.partial(
        pl.kernel,
        out_type=jax.ShapeDtypeStruct((N_ATOMS, NUM_CLASSES), jnp.float32),
        mesh=mesh,
        scratch_types=[
            pltpu.VMEM((IMAP_PAD,), jnp.int32),
            pltpu.VMEM((R,), jnp.int32),
            pltpu.VMEM((R,), jnp.int32),
            pltpu.VMEM((R, NUM_CLASSES), jnp.float32),
        ],
        compiler_params=pltpu.CompilerParams(
            needs_layout_passes=False, use_tc_tiling_on_sc=True),
    )
    def sc_kernel(a_hbm, imap_hbm, out_hbm, imap_v, a_v, col_save, buf):
        wid = lax.axis_index("s") * nc + lax.axis_index("c")
        pltpu.sync_copy(imap_hbm, imap_v)

        zeros16 = jnp.zeros((L,), jnp.float32)
        ones16 = jnp.full((L,), 1.0, jnp.float32)
        row16 = lax.iota(jnp.int32, L)

        tail_cols = row16 + (NUM_CLASSES // L) * L
        tail_mask = tail_cols < NUM_CLASSES

        def zero_body(r, carry):
            for j in range(NUM_CLASSES // L):
                buf[r, pl.ds(j * L, L)] = zeros16
            plsc.store_scatter(
                buf, [jnp.full((L,), r, jnp.int32), tail_cols], zeros16,
                mask=tail_mask)
            return carry

        lax.fori_loop(0, R, zero_body, 0)

        niter = (G - wid + nw - 1) // nw

        def chunk_body(t, carry):
            g = wid + t * nw
            pltpu.sync_copy(a_hbm.at[pl.ds(g * R, R)], a_v)
            for j in range(R // L):
                av = a_v[pl.ds(j * L, L)]
                mapped = plsc.load_gather(imap_v, [av])
                plsc.store_scatter(buf, [row16 + (j * L), mapped], ones16)
                col_save[pl.ds(j * L, L)] = mapped
            pltpu.sync_copy(buf, out_hbm.at[pl.ds(g * R, R)])
            for j in range(R // L):
                mapped = col_save[pl.ds(j * L, L)]
                plsc.store_scatter(buf, [row16 + (j * L), mapped], zeros16)
            return carry

        lax.fori_loop(0, niter, chunk_body, 0)

    return sc_kernel(atomic_numbers, imap_pad)


# speedup vs baseline: 20.3544x; 1.6927x over previous
import functools

import jax
import jax.numpy as jnp
from jax import lax
from jax.experimental import pallas as pl
from jax.experimental.pallas import tpu as pltpu
from jax.experimental.pallas import tpu_sc as plsc

NUM_CLASSES = 119
N_ATOMS = 100000
L = 16
C = 128
NFULL = N_ATOMS // C
TAIL = N_ATOMS - NFULL * C
IMAP_PAD = 128


def kernel(atomic_numbers, index_map):
    info = plsc.get_sparse_core_info()
    nc, ns = info.num_cores, info.num_subcores
    nw = nc * ns

    imap_pad = jnp.pad(index_map, (0, IMAP_PAD - NUM_CLASSES))

    mesh = plsc.VectorSubcoreMesh(core_axis_name="c", subcore_axis_name="s")

    @functools.partial(
        pl.kernel,
        out_type=jax.ShapeDtypeStruct((NUM_CLASSES, N_ATOMS), jnp.float32),
        mesh=mesh,
        scratch_types=[
            pltpu.VMEM((IMAP_PAD,), jnp.int32),
            pltpu.VMEM((C,), jnp.int32),
            pltpu.VMEM((C,), jnp.int32),
            pltpu.VMEM((NUM_CLASSES, C), jnp.float32),
            pltpu.VMEM((NUM_CLASSES, TAIL), jnp.float32),
        ],
        compiler_params=pltpu.CompilerParams(needs_layout_passes=False),
    )
    def sc_kernel(a_hbm, imap_hbm, out_hbm, imap_v, a_v, m_v, buf, buf_t):
        wid = lax.axis_index("s") * nc + lax.axis_index("c")
        pltpu.sync_copy(imap_hbm, imap_v)

        zeros16 = jnp.zeros((L,), jnp.float32)
        ones16 = jnp.full((L,), 1.0, jnp.float32)
        iota16 = lax.iota(jnp.int32, L)

        def zero_body(r, carry):
            for j in range(C // L):
                buf[r, pl.ds(j * L, L)] = zeros16
            return carry

        lax.fori_loop(0, NUM_CLASSES, zero_body, 0)

        niter = (NFULL - wid + nw - 1) // nw

        def chunk_body(t, carry):
            g = wid + t * nw
            pltpu.sync_copy(a_hbm.at[pl.ds(g * C, C)], a_v)
            for j in range(C // L):
                av = a_v[pl.ds(j * L, L)]
                mapped = plsc.load_gather(imap_v, [av])
                plsc.store_scatter(buf, [mapped, iota16 + j * L], ones16)
                m_v[pl.ds(j * L, L)] = mapped
            pltpu.sync_copy(buf, out_hbm.at[:, pl.ds(g * C, C)])
            for j in range(C // L):
                mapped = m_v[pl.ds(j * L, L)]
                plsc.store_scatter(buf, [mapped, iota16 + j * L], zeros16)
            return carry

        lax.fori_loop(0, niter, chunk_body, 0)

        @pl.when(wid == NFULL % nw)
        def _():
            def tail_zero_body(r, carry):
                for j in range(TAIL // L):
                    buf_t[r, pl.ds(j * L, L)] = zeros16
                return carry

            lax.fori_loop(0, NUM_CLASSES, tail_zero_body, 0)
            pltpu.sync_copy(a_hbm.at[pl.ds(NFULL * C, TAIL)], a_v.at[pl.ds(0, TAIL)])
            for j in range(TAIL // L):
                av = a_v[pl.ds(j * L, L)]
                mapped = plsc.load_gather(imap_v, [av])
                plsc.store_scatter(buf_t, [mapped, iota16 + j * L], ones16)
            pltpu.sync_copy(buf_t, out_hbm.at[:, pl.ds(NFULL * C, TAIL)])

    return sc_kernel(atomic_numbers, imap_pad).T
